# baseline (speedup 1.0000x reference)
import functools

import jax
import jax.numpy as jnp
from jax import lax
from jax.experimental import pallas as pl
from jax.experimental.pallas import tpu as pltpu
from jax.experimental.pallas import tpu_sc as plsc

NC = 2
NS = 16
NQ = 4
B = 128
HQ = 32
EPS = 1e-7


def _stage_a_body(x_ref, lg_ref, lb_ref, h4_ref):
    x = x_ref[...]
    mu = jnp.mean(x, axis=-1, keepdims=True)
    var = jnp.mean((x - mu) ** 2, axis=-1, keepdims=True)
    h = (x - mu) * lax.rsqrt(var + 1e-5) * lg_ref[...] + lb_ref[...]
    h = jnp.maximum(h, 0.0)
    for q in range(NQ):
        h4_ref[q] = h[:, q * HQ:(q + 1) * HQ]


def _stage_a(x, lg, lb):
    n, d = x.shape
    bn = 2000
    return pl.pallas_call(
        _stage_a_body,
        grid=(n // bn,),
        in_specs=[
            pl.BlockSpec((bn, d), lambda i: (i, 0)),
            pl.BlockSpec((1, d), lambda i: (0, 0)),
            pl.BlockSpec((1, d), lambda i: (0, 0)),
        ],
        out_specs=pl.BlockSpec((NQ, bn, HQ), lambda i: (0, i, 0)),
        out_shape=jax.ShapeDtypeStruct((NQ, n, HQ), jnp.float32),
    )(x, lg.reshape(1, d), lb.reshape(1, d))


def _stage_b_body(ea_ref, we_ref, be_ref, out_ref):
    ea = jnp.dot(ea_ref[...], we_ref[...],
                 preferred_element_type=jnp.float32) + be_ref[...]
    for q in range(NQ):
        out_ref[q] = ea[:, q * HQ:(q + 1) * HQ]


def _stage_b(eattr, we, be, e_pad):
    e, ed = eattr.shape
    d = we.shape[1]
    be_blk = 2000
    return pl.pallas_call(
        _stage_b_body,
        grid=(e // be_blk,),
        in_specs=[
            pl.BlockSpec((be_blk, ed), lambda i: (i, 0)),
            pl.BlockSpec((ed, d), lambda i: (0, 0)),
            pl.BlockSpec((1, d), lambda i: (0, 0)),
        ],
        out_specs=pl.BlockSpec((NQ, be_blk, HQ), lambda i: (0, i, 0)),
        out_shape=jax.ShapeDtypeStruct((NQ, e_pad, HQ), jnp.float32),
    )(eattr, we, be.reshape(1, d))


def _layer_norm_blk(x, g, b):
    mu = jnp.mean(x, axis=-1, keepdims=True)
    var = jnp.mean((x - mu) ** 2, axis=-1, keepdims=True)
    return (x - mu) * lax.rsqrt(var + 1e-5) * g + b


def _stage_c_body(s_ref, n_ref, h4_ref, x_ref, w1_ref, b1_ref, mg_ref,
                  mb_ref, w2_ref, b2_ref, o_ref):
    parts = [n_ref[q] / (s_ref[q] + 1e-16) + h4_ref[q] for q in range(NQ)]
    out = jnp.concatenate(parts, axis=-1)
    hid = jnp.dot(out, w1_ref[...],
                  preferred_element_type=jnp.float32) + b1_ref[...]
    hid = jnp.maximum(_layer_norm_blk(hid, mg_ref[...], mb_ref[...]), 0.0)
    y = jnp.dot(hid, w2_ref[...],
                preferred_element_type=jnp.float32) + b2_ref[...]
    o_ref[...] = x_ref[...] + y


def _stage_c(s_acc, n_acc, h4, x, w1, b1, mg, mb, w2, b2):
    n, d = x.shape
    d2 = w1.shape[1]
    bn = 2000
    return pl.pallas_call(
        _stage_c_body,
        grid=(n // bn,),
        in_specs=[
            pl.BlockSpec((NQ, bn, HQ), lambda i: (0, i, 0)),
            pl.BlockSpec((NQ, bn, HQ), lambda i: (0, i, 0)),
            pl.BlockSpec((NQ, bn, HQ), lambda i: (0, i, 0)),
            pl.BlockSpec((bn, d), lambda i: (i, 0)),
            pl.BlockSpec((d, d2), lambda i: (0, 0)),
            pl.BlockSpec((1, d2), lambda i: (0, 0)),
            pl.BlockSpec((1, d2), lambda i: (0, 0)),
            pl.BlockSpec((1, d2), lambda i: (0, 0)),
            pl.BlockSpec((d2, d), lambda i: (0, 0)),
            pl.BlockSpec((1, d), lambda i: (0, 0)),
        ],
        out_specs=pl.BlockSpec((bn, d), lambda i: (i, 0)),
        out_shape=jax.ShapeDtypeStruct((n, d), jnp.float32),
    )(s_acc, n_acc, h4, x, w1, b1.reshape(1, d2), mg.reshape(1, d2),
      mb.reshape(1, d2), w2, b2.reshape(1, d))


def _gelu_exact(x):
    return 0.5 * x * (1.0 + lax.erf(x * 0.7071067811865476))


def _stage_d_body(x_ref, w1_ref, b1_ref, g_ref, beta_ref, w2_ref, b2_ref,
                  w3_ref, b3_ref, w4_ref, b4_ref, w5_ref, b5_ref, w6_ref,
                  b6_ref, o_ref):
    t = jnp.dot(x_ref[...], w1_ref[...],
                preferred_element_type=jnp.float32) + b1_ref[...]
    t = _layer_norm_blk(t, g_ref[...], beta_ref[...])
    t = _gelu_exact(t)
    t = _gelu_exact(jnp.dot(t, w2_ref[...],
                            preferred_element_type=jnp.float32) + b2_ref[...])
    t = _gelu_exact(jnp.dot(t, w3_ref[...],
                            preferred_element_type=jnp.float32) + b3_ref[...])
    t = _gelu_exact(jnp.dot(t, w4_ref[...],
                            preferred_element_type=jnp.float32) + b4_ref[...])
    t = _gelu_exact(jnp.dot(t, w5_ref[...],
                            preferred_element_type=jnp.float32) + b5_ref[...])
    o_ref[...] = jnp.dot(t, w6_ref[...],
                         preferred_element_type=jnp.float32) + b6_ref[...]


def _stage_d(x, m):
    n, d = x.shape
    bn = 1000
    w6 = m['W6']
    d6in, d6out = w6.shape
    dpad = 128
    w6p = jnp.zeros((d6in, dpad), jnp.float32).at[:, :d6out].set(w6)
    b6p = jnp.zeros((dpad,), jnp.float32).at[:d6out].set(m['b6'])
    ws = [m['W1'], m['W2'], m['W3'], m['W4'], m['W5'], w6p]
    bs = [m['b1'], m['b2'], m['b3'], m['b4'], m['b5'], b6p]
    gs = [m['g'], m['beta']]
    in_specs = [pl.BlockSpec((bn, d), lambda i: (i, 0))]
    args = [x]

    def add_mat(w):
        in_specs.append(pl.BlockSpec(w.shape, lambda i: (0, 0)))
        args.append(w)

    def add_vec(v):
        in_specs.append(pl.BlockSpec((1, v.shape[0]), lambda i: (0, 0)))
        args.append(v.reshape(1, -1))

    add_mat(ws[0]); add_vec(bs[0]); add_vec(gs[0]); add_vec(gs[1])
    for w, b in zip(ws[1:], bs[1:]):
        add_mat(w); add_vec(b)
    out = pl.pallas_call(
        _stage_d_body,
        grid=(n // bn,),
        in_specs=in_specs,
        out_specs=pl.BlockSpec((bn, dpad), lambda i: (i, 0)),
        out_shape=jax.ShapeDtypeStruct((n, dpad), jnp.float32),
    )(*args)
    return out[:, :d6out]


def _sc_edge_body(acc_rows, nblk, epw,
                  h4n, ea4, sidx, didx, t16, s_out, n_out,
                  src_v, dst_v, r0, r1, r2, a0, a1, a2, tv_v, s_acc, n_acc,
                  g0, g1, g2, s0, s1, s2):
    rows = (r0, r1, r2)
    eav = (a0, a1, a2)
    gsem = (g0, g1, g2)
    ssem = (s0, s1, s2)
    c = lax.axis_index("c")
    sid = lax.axis_index("s")
    stripe = acc_rows // NS
    zeros16 = jnp.zeros((16,), jnp.float32)

    pltpu.sync_copy(didx.at[sid], dst_v)
    pltpu.sync_copy(t16, tv_v)
    tv = tv_v[...]
    edge_base = sid * epw

    for p in range(2):
        q = c * 2 + p

        def zrow(r, carry):
            for k in range(HQ // 16):
                rows[0][r, pl.ds(k * 16, 16)] = zeros16
            return carry
        lax.fori_loop(0, B, zrow, 0, unroll=4)

        def zchunk(k, carry):
            base = sid * stripe + k * B
            pltpu.sync_copy(rows[0], s_acc.at[pl.ds(base, B)])
            pltpu.sync_copy(rows[0], n_acc.at[pl.ds(base, B)])
            return carry
        lax.fori_loop(0, stripe // B, zchunk, 0)

        pltpu.sync_copy(sidx.at[c, p, sid], src_v)
        plsc.subcore_barrier()

        def issue_in(j, b):
            pltpu.async_copy(h4n.at[src_v.at[j]], rows[b], gsem[b])
            pltpu.async_copy(ea4.at[q, pl.ds(edge_base + j * B, B)],
                             eav[b], gsem[b])

        def wait_in(b):
            for _ in range(2):
                pltpu.make_async_copy(ea4.at[q, pl.ds(0, B)], rows[b],
                                      gsem[b]).wait()

        def compute(b):
            rb, ab = rows[b], eav[b]
            nch = HQ // 16
            u_rows = 8

            def rowfn(i, carry):
                r0 = i * u_rows
                sls = [(r0 + u, pl.ds(k * 16, 16))
                       for u in range(u_rows) for k in range(nch)]
                hv = [rb[r, sl] for r, sl in sls]
                av = [ab[r, sl] for r, sl in sls]
                ms = [jnp.maximum(h + a, 0.0) + EPS
                      for h, a in zip(hv, av)]
                es = [jnp.exp(m * tv) for m in ms]
                mes = [m * e for m, e in zip(ms, es)]
                for (r, sl), e in zip(sls, es):
                    rb[r, sl] = e
                for (r, sl), me in zip(sls, mes):
                    ab[r, sl] = me
                return carry
            lax.fori_loop(0, B // u_rows, rowfn, 0)

        def issue_scatter(j, b):
            pltpu.async_copy(rows[b], s_acc.at[dst_v.at[j]], ssem[b],
                             add=True)
            pltpu.async_copy(eav[b], n_acc.at[dst_v.at[j]], ssem[b],
                             add=True)

        def wait_scatter(b):
            for _ in range(2):
                pltpu.make_async_copy(ea4.at[q, pl.ds(0, B)], rows[b],
                                      ssem[b]).wait()

        issue_in(0, 0)
        issue_in(1, 1)
        wait_in(0)
        compute(0)
        issue_scatter(0, 0)
        issue_in(2, 2)

        @pl.loop(1, nblk - 2, step=3)
        def _main(j0):
            for db in range(3):
                j = j0 + db
                b = (1 + db) % 3
                wait_in(b)
                compute(b)
                issue_scatter(j, b)
                bp = db % 3
                wait_scatter(bp)
                issue_in(j + 2, bp)

        wait_in(1)
        compute(1)
        issue_scatter(nblk - 2, 1)
        wait_in(2)
        compute(2)
        issue_scatter(nblk - 1, 2)
        for b in range(3):
            wait_scatter(b)
        plsc.subcore_barrier()

        pltpu.sync_copy(s_acc.at[pl.ds(sid * stripe, stripe)],
                        s_out.at[q, pl.ds(sid * stripe, stripe)])
        pltpu.sync_copy(n_acc.at[pl.ds(sid * stripe, stripe)],
                        n_out.at[q, pl.ds(sid * stripe, stripe)])


def _sc_edge(h4n, ea4, sidx, didx, t16, acc_rows, nblk, epw):
    mesh = plsc.VectorSubcoreMesh(core_axis_name="c", subcore_axis_name="s")
    body = functools.partial(_sc_edge_body, acc_rows, nblk, epw)
    f = pl.kernel(
        body,
        out_type=[
            jax.ShapeDtypeStruct((NQ, acc_rows, HQ), jnp.float32),
            jax.ShapeDtypeStruct((NQ, acc_rows, HQ), jnp.float32),
        ],
        mesh=mesh,
        compiler_params=pltpu.CompilerParams(use_tc_tiling_on_sc=False),
        scratch_types=[
            pltpu.VMEM((nblk, B), jnp.int32),
            pltpu.VMEM((nblk, B), jnp.int32),
            pltpu.VMEM((B, HQ), jnp.float32),
            pltpu.VMEM((B, HQ), jnp.float32),
            pltpu.VMEM((B, HQ), jnp.float32),
            pltpu.VMEM((B, HQ), jnp.float32),
            pltpu.VMEM((B, HQ), jnp.float32),
            pltpu.VMEM((B, HQ), jnp.float32),
            pltpu.VMEM((16,), jnp.float32),
            pltpu.VMEM_SHARED((acc_rows, HQ), jnp.float32),
            pltpu.VMEM_SHARED((acc_rows, HQ), jnp.float32),
            pltpu.SemaphoreType.DMA,
            pltpu.SemaphoreType.DMA,
            pltpu.SemaphoreType.DMA,
            pltpu.SemaphoreType.DMA,
            pltpu.SemaphoreType.DMA,
            pltpu.SemaphoreType.DMA,
        ],
    )
    return f(h4n, ea4, sidx, didx, t16)


def kernel(node_features, edge_index, edge_features, params):
    x = node_features
    n, d = x.shape
    e, ed = edge_features.shape

    nblk = (e + NS * B - 1) // (NS * B)
    nblk = ((nblk + 2) // 3) * 3
    epw = nblk * B
    e_pad = NS * epw
    acc_rows = ((n + NS * B - 1) // (NS * B)) * (NS * B)
    trash = acc_rows - 1

    src = edge_index[0]
    dst = edge_index[1]
    pad = e_pad - e
    src_pad = jnp.concatenate([src, jnp.zeros((pad,), jnp.int32)])
    dst_pad = jnp.concatenate([dst, jnp.full((pad,), trash, jnp.int32)])
    sidx = jnp.stack([jnp.stack([src_pad + (2 * c + p) * n for p in range(2)])
                      for c in range(NC)]).reshape(NC, 2, NS, nblk, B)
    didx = dst_pad.reshape(NS, nblk, B)

    stacked = jax.tree.map(lambda *xs: jnp.stack(xs), *params['gcn'])

    def layer_step(xc, p):
        h4 = _stage_a(xc, p['lg'], p['lb'])
        ea4 = _stage_b(edge_features, p['We'], p['be'], e_pad)
        t16 = jnp.broadcast_to(p['t'], (16,)).astype(jnp.float32)
        s_acc, n_acc = _sc_edge(h4.reshape(NQ * n, HQ), ea4, sidx, didx,
                                t16, acc_rows, nblk, epw)
        xc = _stage_c(s_acc, n_acc, h4, xc, p['W1'], p['b1'], p['mg'],
                      p['mb'], p['W2'], p['b2'])
        return xc, None

    x, _ = lax.scan(layer_step, x, stacked)

    return _stage_d(x, params['mlp'])


# speedup vs baseline: 1.7477x; 1.7477x over previous
import functools

import jax
import jax.numpy as jnp
from jax import lax
from jax.experimental import pallas as pl
from jax.experimental.pallas import tpu as pltpu
from jax.experimental.pallas import tpu_sc as plsc

NC = 2
NS = 16
NQ = 4
B = 128
HQ = 32
EPS = 1e-7


def _stage_a_body(x_ref, lg_ref, lb_ref, h4_ref):
    x = x_ref[...]
    mu = jnp.mean(x, axis=-1, keepdims=True)
    var = jnp.mean((x - mu) ** 2, axis=-1, keepdims=True)
    h = (x - mu) * lax.rsqrt(var + 1e-5) * lg_ref[...] + lb_ref[...]
    h = jnp.maximum(h, 0.0)
    for q in range(NQ):
        h4_ref[q] = h[:, q * HQ:(q + 1) * HQ]


def _stage_a(x, lg, lb):
    n, d = x.shape
    bn = 2000
    return pl.pallas_call(
        _stage_a_body,
        grid=(n // bn,),
        in_specs=[
            pl.BlockSpec((bn, d), lambda i: (i, 0)),
            pl.BlockSpec((1, d), lambda i: (0, 0)),
            pl.BlockSpec((1, d), lambda i: (0, 0)),
        ],
        out_specs=pl.BlockSpec((NQ, bn, HQ), lambda i: (0, i, 0)),
        out_shape=jax.ShapeDtypeStruct((NQ, n, HQ), jnp.float32),
    )(x, lg.reshape(1, d), lb.reshape(1, d))


def _stage_b_body(ea_ref, wp_ref, bp_ref, out_ref):
    ea = jnp.dot(ea_ref[...], wp_ref[...],
                 preferred_element_type=jnp.float32) + bp_ref[...]
    for q in range(NQ):
        out_ref[q] = ea[:, q * 4 * HQ:(q + 1) * 4 * HQ]


def _stage_b(eattr_r, w_pack, b_pack, e_pad):
    e4, ed4 = eattr_r.shape
    dp = w_pack.shape[1]
    be_blk = 800
    return pl.pallas_call(
        _stage_b_body,
        grid=(e4 // be_blk,),
        in_specs=[
            pl.BlockSpec((be_blk, ed4), lambda i: (i, 0)),
            pl.BlockSpec((ed4, dp), lambda i: (0, 0)),
            pl.BlockSpec((1, dp), lambda i: (0, 0)),
        ],
        out_specs=pl.BlockSpec((NQ, be_blk, 4 * HQ), lambda i: (0, i, 0)),
        out_shape=jax.ShapeDtypeStruct((NQ, e_pad // 4, 4 * HQ),
                                       jnp.float32),
    )(eattr_r, w_pack, b_pack.reshape(1, dp))


def _layer_norm_blk(x, g, b):
    mu = jnp.mean(x, axis=-1, keepdims=True)
    var = jnp.mean((x - mu) ** 2, axis=-1, keepdims=True)
    return (x - mu) * lax.rsqrt(var + 1e-5) * g + b


def _stage_c_body(s_ref, n_ref, h4_ref, x_ref, w1_ref, b1_ref, mg_ref,
                  mb_ref, w2_ref, b2_ref, o_ref):
    parts = [n_ref[q] / (s_ref[q] + 1e-16) + h4_ref[q] for q in range(NQ)]
    out = jnp.concatenate(parts, axis=-1)
    hid = jnp.dot(out, w1_ref[...],
                  preferred_element_type=jnp.float32) + b1_ref[...]
    hid = jnp.maximum(_layer_norm_blk(hid, mg_ref[...], mb_ref[...]), 0.0)
    y = jnp.dot(hid, w2_ref[...],
                preferred_element_type=jnp.float32) + b2_ref[...]
    o_ref[...] = x_ref[...] + y


def _stage_c(s_acc, n_acc, h4, x, w1, b1, mg, mb, w2, b2):
    n, d = x.shape
    d2 = w1.shape[1]
    bn = 2000
    return pl.pallas_call(
        _stage_c_body,
        grid=(n // bn,),
        in_specs=[
            pl.BlockSpec((NQ, bn, HQ), lambda i: (0, i, 0)),
            pl.BlockSpec((NQ, bn, HQ), lambda i: (0, i, 0)),
            pl.BlockSpec((NQ, bn, HQ), lambda i: (0, i, 0)),
            pl.BlockSpec((bn, d), lambda i: (i, 0)),
            pl.BlockSpec((d, d2), lambda i: (0, 0)),
            pl.BlockSpec((1, d2), lambda i: (0, 0)),
            pl.BlockSpec((1, d2), lambda i: (0, 0)),
            pl.BlockSpec((1, d2), lambda i: (0, 0)),
            pl.BlockSpec((d2, d), lambda i: (0, 0)),
            pl.BlockSpec((1, d), lambda i: (0, 0)),
        ],
        out_specs=pl.BlockSpec((bn, d), lambda i: (i, 0)),
        out_shape=jax.ShapeDtypeStruct((n, d), jnp.float32),
    )(s_acc, n_acc, h4, x, w1, b1.reshape(1, d2), mg.reshape(1, d2),
      mb.reshape(1, d2), w2, b2.reshape(1, d))


def _gelu_exact(x):
    return 0.5 * x * (1.0 + lax.erf(x * 0.7071067811865476))


def _stage_d_body(x_ref, w1_ref, b1_ref, g_ref, beta_ref, w2_ref, b2_ref,
                  w3_ref, b3_ref, w4_ref, b4_ref, w5_ref, b5_ref, w6_ref,
                  b6_ref, o_ref):
    t = jnp.dot(x_ref[...], w1_ref[...],
                preferred_element_type=jnp.float32) + b1_ref[...]
    t = _layer_norm_blk(t, g_ref[...], beta_ref[...])
    t = _gelu_exact(t)
    t = _gelu_exact(jnp.dot(t, w2_ref[...],
                            preferred_element_type=jnp.float32) + b2_ref[...])
    t = _gelu_exact(jnp.dot(t, w3_ref[...],
                            preferred_element_type=jnp.float32) + b3_ref[...])
    t = _gelu_exact(jnp.dot(t, w4_ref[...],
                            preferred_element_type=jnp.float32) + b4_ref[...])
    t = _gelu_exact(jnp.dot(t, w5_ref[...],
                            preferred_element_type=jnp.float32) + b5_ref[...])
    o_ref[...] = jnp.dot(t, w6_ref[...],
                         preferred_element_type=jnp.float32) + b6_ref[...]


def _stage_d(x, m):
    n, d = x.shape
    bn = 1000
    w6 = m['W6']
    d6in, d6out = w6.shape
    dpad = 128
    w6p = jnp.zeros((d6in, dpad), jnp.float32).at[:, :d6out].set(w6)
    b6p = jnp.zeros((dpad,), jnp.float32).at[:d6out].set(m['b6'])
    ws = [m['W1'], m['W2'], m['W3'], m['W4'], m['W5'], w6p]
    bs = [m['b1'], m['b2'], m['b3'], m['b4'], m['b5'], b6p]
    gs = [m['g'], m['beta']]
    in_specs = [pl.BlockSpec((bn, d), lambda i: (i, 0))]
    args = [x]

    def add_mat(w):
        in_specs.append(pl.BlockSpec(w.shape, lambda i: (0, 0)))
        args.append(w)

    def add_vec(v):
        in_specs.append(pl.BlockSpec((1, v.shape[0]), lambda i: (0, 0)))
        args.append(v.reshape(1, -1))

    add_mat(ws[0]); add_vec(bs[0]); add_vec(gs[0]); add_vec(gs[1])
    for w, b in zip(ws[1:], bs[1:]):
        add_mat(w); add_vec(b)
    out = pl.pallas_call(
        _stage_d_body,
        grid=(n // bn,),
        in_specs=in_specs,
        out_specs=pl.BlockSpec((bn, dpad), lambda i: (i, 0)),
        out_shape=jax.ShapeDtypeStruct((n, dpad), jnp.float32),
    )(*args)
    return out[:, :d6out]


def _sc_edge_body(acc_rows, nblk, epw,
                  h4n, ea4, sidx, didx, t16, s_out, n_out,
                  src_v, dst_v, r0, r1, r2, a0, a1, a2, m0, m1, m2,
                  tv_v, s_acc, n_acc,
                  g0, g1, g2, s0, s1, s2):
    rows = (r0, r1, r2)
    eav = (a0, a1, a2)
    mev = (m0, m1, m2)
    gsem = (g0, g1, g2)
    ssem = (s0, s1, s2)
    c = lax.axis_index("c")
    sid = lax.axis_index("s")
    stripe = acc_rows // NS
    zeros16 = jnp.zeros((16,), jnp.float32)

    pltpu.sync_copy(didx.at[sid], dst_v)
    pltpu.sync_copy(t16, tv_v)
    tv = tv_v[...]
    edge_base = sid * epw

    for p in range(2):
        q = c * 2 + p

        def zrow(r, carry):
            for k in range(HQ // 16):
                rows[0][r, pl.ds(k * 16, 16)] = zeros16
            return carry
        lax.fori_loop(0, B, zrow, 0, unroll=4)

        def zchunk(k, carry):
            base = sid * stripe + k * B
            pltpu.sync_copy(rows[0], s_acc.at[pl.ds(base, B)])
            pltpu.sync_copy(rows[0], n_acc.at[pl.ds(base, B)])
            return carry
        lax.fori_loop(0, stripe // B, zchunk, 0)

        pltpu.sync_copy(sidx.at[c, p, sid], src_v)
        plsc.subcore_barrier()

        def issue_in(j, b):
            pltpu.async_copy(h4n.at[src_v.at[j]], rows[b], gsem[b])
            pltpu.async_copy(
                ea4.at[q, pl.ds((edge_base + j * B) // 4, B // 4)],
                eav[b], gsem[b])

        def wait_in(b):
            for _ in range(2):
                pltpu.make_async_copy(ea4.at[q, pl.ds(0, B // 4)], eav[b],
                                      gsem[b]).wait()

        def compute(b):
            rb, ab, mb_ = rows[b], eav[b], mev[b]
            nch = HQ // 16

            def rowfn(g, carry):
                r0 = g * 4
                sls = [(r0 + u, pl.ds(k * 16, 16), pl.ds(u * HQ + k * 16, 16))
                       for u in range(4) for k in range(nch)]
                hv = [rb[r, sl] for r, sl, _ in sls]
                av = [ab[g, asl] for _, _, asl in sls]
                ms = [jnp.maximum(h + a, 0.0) + EPS
                      for h, a in zip(hv, av)]
                es = [jnp.exp(m * tv) for m in ms]
                mes = [m * e for m, e in zip(ms, es)]
                for (r, sl, _), e in zip(sls, es):
                    rb[r, sl] = e
                for (r, sl, _), me in zip(sls, mes):
                    mb_[r, sl] = me
                return carry
            lax.fori_loop(0, B // 4, rowfn, 0)

        def issue_scatter(j, b):
            pltpu.async_copy(rows[b], s_acc.at[dst_v.at[j]], ssem[b],
                             add=True)
            pltpu.async_copy(mev[b], n_acc.at[dst_v.at[j]], ssem[b],
                             add=True)

        def wait_scatter(b):
            for _ in range(2):
                pltpu.make_async_copy(ea4.at[q, pl.ds(0, B // 4)], eav[b],
                                      ssem[b]).wait()

        issue_in(0, 0)
        issue_in(1, 1)
        wait_in(0)
        compute(0)
        issue_scatter(0, 0)
        issue_in(2, 2)

        @pl.loop(1, nblk - 2, step=3)
        def _main(j0):
            for db in range(3):
                j = j0 + db
                b = (1 + db) % 3
                wait_in(b)
                compute(b)
                issue_scatter(j, b)
                bp = db % 3
                wait_scatter(bp)
                issue_in(j + 2, bp)

        wait_in(1)
        compute(1)
        issue_scatter(nblk - 2, 1)
        wait_in(2)
        compute(2)
        issue_scatter(nblk - 1, 2)
        for b in range(3):
            wait_scatter(b)
        plsc.subcore_barrier()

        pltpu.sync_copy(s_acc.at[pl.ds(sid * stripe, stripe)],
                        s_out.at[q, pl.ds(sid * stripe, stripe)])
        pltpu.sync_copy(n_acc.at[pl.ds(sid * stripe, stripe)],
                        n_out.at[q, pl.ds(sid * stripe, stripe)])


def _sc_edge(h4n, ea4, sidx, didx, t16, acc_rows, nblk, epw):
    mesh = plsc.VectorSubcoreMesh(core_axis_name="c", subcore_axis_name="s")
    body = functools.partial(_sc_edge_body, acc_rows, nblk, epw)
    f = pl.kernel(
        body,
        out_type=[
            jax.ShapeDtypeStruct((NQ, acc_rows, HQ), jnp.float32),
            jax.ShapeDtypeStruct((NQ, acc_rows, HQ), jnp.float32),
        ],
        mesh=mesh,
        compiler_params=pltpu.CompilerParams(use_tc_tiling_on_sc=False),
        scratch_types=[
            pltpu.VMEM((nblk, B), jnp.int32),
            pltpu.VMEM((nblk, B), jnp.int32),
            pltpu.VMEM((B, HQ), jnp.float32),
            pltpu.VMEM((B, HQ), jnp.float32),
            pltpu.VMEM((B, HQ), jnp.float32),
            pltpu.VMEM((B // 4, 4 * HQ), jnp.float32),
            pltpu.VMEM((B // 4, 4 * HQ), jnp.float32),
            pltpu.VMEM((B // 4, 4 * HQ), jnp.float32),
            pltpu.VMEM((B, HQ), jnp.float32),
            pltpu.VMEM((B, HQ), jnp.float32),
            pltpu.VMEM((B, HQ), jnp.float32),
            pltpu.VMEM((16,), jnp.float32),
            pltpu.VMEM_SHARED((acc_rows, HQ), jnp.float32),
            pltpu.VMEM_SHARED((acc_rows, HQ), jnp.float32),
            pltpu.SemaphoreType.DMA,
            pltpu.SemaphoreType.DMA,
            pltpu.SemaphoreType.DMA,
            pltpu.SemaphoreType.DMA,
            pltpu.SemaphoreType.DMA,
            pltpu.SemaphoreType.DMA,
        ],
    )
    return f(h4n, ea4, sidx, didx, t16)


def kernel(node_features, edge_index, edge_features, params):
    x = node_features
    n, d = x.shape
    e, ed = edge_features.shape

    nblk = (e + NS * B - 1) // (NS * B)
    nblk = ((nblk + 2) // 3) * 3
    epw = nblk * B
    e_pad = NS * epw
    acc_rows = ((n + NS * B - 1) // (NS * B)) * (NS * B)
    trash = acc_rows - 1

    src = edge_index[0]
    dst = edge_index[1]
    pad = e_pad - e
    src_pad = jnp.concatenate([src, jnp.zeros((pad,), jnp.int32)])
    dst_pad = jnp.concatenate([dst, jnp.full((pad,), trash, jnp.int32)])
    sidx = jnp.stack([jnp.stack([src_pad + (2 * c + p) * n for p in range(2)])
                      for c in range(NC)]).reshape(NC, 2, NS, nblk, B)
    didx = dst_pad.reshape(NS, nblk, B)

    stacked = jax.tree.map(lambda *xs: jnp.stack(xs), *params['gcn'])

    eattr_r = edge_features.reshape(e // 4, 4 * ed)

    def layer_step(xc, p):
        h4 = _stage_a(xc, p['lg'], p['lb'])
        we = p['We']
        w_pack = jnp.zeros((4 * ed, NQ * 4 * HQ), jnp.float32)
        b_pack = jnp.zeros((NQ * 4 * HQ,), jnp.float32)
        for u in range(4):
            for qq in range(NQ):
                w_pack = w_pack.at[
                    u * ed:(u + 1) * ed,
                    qq * 4 * HQ + u * HQ:qq * 4 * HQ + (u + 1) * HQ].set(
                        we[:, qq * HQ:(qq + 1) * HQ])
                b_pack = b_pack.at[
                    qq * 4 * HQ + u * HQ:qq * 4 * HQ + (u + 1) * HQ].set(
                        p['be'][qq * HQ:(qq + 1) * HQ])
        ea4 = _stage_b(eattr_r, w_pack, b_pack, e_pad)
        t16 = jnp.broadcast_to(p['t'], (16,)).astype(jnp.float32)
        s_acc, n_acc = _sc_edge(h4.reshape(NQ * n, HQ), ea4, sidx, didx,
                                t16, acc_rows, nblk, epw)
        xc = _stage_c(s_acc, n_acc, h4, xc, p['W1'], p['b1'], p['mg'],
                      p['mb'], p['W2'], p['b2'])
        return xc, None

    x, _ = lax.scan(layer_step, x, stacked)

    return _stage_d(x, params['mlp'])


# speedup vs baseline: 1.8544x; 1.0611x over previous
import functools

import jax
import jax.numpy as jnp
from jax import lax
from jax.experimental import pallas as pl
from jax.experimental.pallas import tpu as pltpu
from jax.experimental.pallas import tpu_sc as plsc

NC = 2
NS = 16
NQ = 4
B = 128
HQ = 32
EPS = 1e-7


def _stage_a_body(x_ref, lg_ref, lb_ref, h4_ref):
    x = x_ref[...]
    mu = jnp.mean(x, axis=-1, keepdims=True)
    var = jnp.mean((x - mu) ** 2, axis=-1, keepdims=True)
    h = (x - mu) * lax.rsqrt(var + 1e-5) * lg_ref[...] + lb_ref[...]
    h = jnp.maximum(h, 0.0)
    for q in range(NQ):
        h4_ref[q] = h[:, q * HQ:(q + 1) * HQ]


def _stage_a(x, lg, lb):
    n, d = x.shape
    bn = 2000
    return pl.pallas_call(
        _stage_a_body,
        grid=(n // bn,),
        in_specs=[
            pl.BlockSpec((bn, d), lambda i: (i, 0)),
            pl.BlockSpec((1, d), lambda i: (0, 0)),
            pl.BlockSpec((1, d), lambda i: (0, 0)),
        ],
        out_specs=pl.BlockSpec((NQ, bn, HQ), lambda i: (0, i, 0)),
        out_shape=jax.ShapeDtypeStruct((NQ, n, HQ), jnp.float32),
    )(x, lg.reshape(1, d), lb.reshape(1, d))


def _stage_b_body(ea_ref, wp_ref, bp_ref, out_ref):
    ea = jnp.dot(ea_ref[...], wp_ref[...],
                 preferred_element_type=jnp.float32) + bp_ref[...]
    for q in range(NQ):
        out_ref[q] = ea[:, q * 4 * HQ:(q + 1) * 4 * HQ]


def _stage_b(eattr_r, w_pack, b_pack, e_pad):
    e4, ed4 = eattr_r.shape
    dp = w_pack.shape[1]
    be_blk = 3200
    return pl.pallas_call(
        _stage_b_body,
        grid=(e4 // be_blk,),
        in_specs=[
            pl.BlockSpec((be_blk, ed4), lambda i: (i, 0)),
            pl.BlockSpec((ed4, dp), lambda i: (0, 0)),
            pl.BlockSpec((1, dp), lambda i: (0, 0)),
        ],
        out_specs=pl.BlockSpec((NQ, be_blk, 4 * HQ), lambda i: (0, i, 0)),
        out_shape=jax.ShapeDtypeStruct((NQ, e_pad // 4, 4 * HQ),
                                       jnp.float32),
    )(eattr_r, w_pack, b_pack.reshape(1, dp))


def _layer_norm_blk(x, g, b):
    mu = jnp.mean(x, axis=-1, keepdims=True)
    var = jnp.mean((x - mu) ** 2, axis=-1, keepdims=True)
    return (x - mu) * lax.rsqrt(var + 1e-5) * g + b


def _stage_c_body(s_ref, n_ref, h4_ref, x_ref, w1_ref, b1_ref, mg_ref,
                  mb_ref, w2_ref, b2_ref, o_ref):
    parts = [n_ref[q] / (s_ref[q] + 1e-16) + h4_ref[q] for q in range(NQ)]
    out = jnp.concatenate(parts, axis=-1)
    hid = jnp.dot(out, w1_ref[...],
                  preferred_element_type=jnp.float32) + b1_ref[...]
    hid = jnp.maximum(_layer_norm_blk(hid, mg_ref[...], mb_ref[...]), 0.0)
    y = jnp.dot(hid, w2_ref[...],
                preferred_element_type=jnp.float32) + b2_ref[...]
    o_ref[...] = x_ref[...] + y


def _stage_c(s_acc, n_acc, h4, x, w1, b1, mg, mb, w2, b2):
    n, d = x.shape
    d2 = w1.shape[1]
    bn = 2000
    return pl.pallas_call(
        _stage_c_body,
        grid=(n // bn,),
        in_specs=[
            pl.BlockSpec((NQ, bn, HQ), lambda i: (0, i, 0)),
            pl.BlockSpec((NQ, bn, HQ), lambda i: (0, i, 0)),
            pl.BlockSpec((NQ, bn, HQ), lambda i: (0, i, 0)),
            pl.BlockSpec((bn, d), lambda i: (i, 0)),
            pl.BlockSpec((d, d2), lambda i: (0, 0)),
            pl.BlockSpec((1, d2), lambda i: (0, 0)),
            pl.BlockSpec((1, d2), lambda i: (0, 0)),
            pl.BlockSpec((1, d2), lambda i: (0, 0)),
            pl.BlockSpec((d2, d), lambda i: (0, 0)),
            pl.BlockSpec((1, d), lambda i: (0, 0)),
        ],
        out_specs=pl.BlockSpec((bn, d), lambda i: (i, 0)),
        out_shape=jax.ShapeDtypeStruct((n, d), jnp.float32),
    )(s_acc, n_acc, h4, x, w1, b1.reshape(1, d2), mg.reshape(1, d2),
      mb.reshape(1, d2), w2, b2.reshape(1, d))


def _gelu_exact(x):
    return 0.5 * x * (1.0 + lax.erf(x * 0.7071067811865476))


def _stage_d_body(x_ref, w1_ref, b1_ref, g_ref, beta_ref, w2_ref, b2_ref,
                  w3_ref, b3_ref, w4_ref, b4_ref, w5_ref, b5_ref, w6_ref,
                  b6_ref, o_ref):
    t = jnp.dot(x_ref[...], w1_ref[...],
                preferred_element_type=jnp.float32) + b1_ref[...]
    t = _layer_norm_blk(t, g_ref[...], beta_ref[...])
    t = _gelu_exact(t)
    t = _gelu_exact(jnp.dot(t, w2_ref[...],
                            preferred_element_type=jnp.float32) + b2_ref[...])
    t = _gelu_exact(jnp.dot(t, w3_ref[...],
                            preferred_element_type=jnp.float32) + b3_ref[...])
    t = _gelu_exact(jnp.dot(t, w4_ref[...],
                            preferred_element_type=jnp.float32) + b4_ref[...])
    t = _gelu_exact(jnp.dot(t, w5_ref[...],
                            preferred_element_type=jnp.float32) + b5_ref[...])
    o_ref[...] = jnp.dot(t, w6_ref[...],
                         preferred_element_type=jnp.float32) + b6_ref[...]


def _stage_d(x, m):
    n, d = x.shape
    bn = 1000
    w6 = m['W6']
    d6in, d6out = w6.shape
    dpad = 128
    w6p = jnp.zeros((d6in, dpad), jnp.float32).at[:, :d6out].set(w6)
    b6p = jnp.zeros((dpad,), jnp.float32).at[:d6out].set(m['b6'])
    ws = [m['W1'], m['W2'], m['W3'], m['W4'], m['W5'], w6p]
    bs = [m['b1'], m['b2'], m['b3'], m['b4'], m['b5'], b6p]
    gs = [m['g'], m['beta']]
    in_specs = [pl.BlockSpec((bn, d), lambda i: (i, 0))]
    args = [x]

    def add_mat(w):
        in_specs.append(pl.BlockSpec(w.shape, lambda i: (0, 0)))
        args.append(w)

    def add_vec(v):
        in_specs.append(pl.BlockSpec((1, v.shape[0]), lambda i: (0, 0)))
        args.append(v.reshape(1, -1))

    add_mat(ws[0]); add_vec(bs[0]); add_vec(gs[0]); add_vec(gs[1])
    for w, b in zip(ws[1:], bs[1:]):
        add_mat(w); add_vec(b)
    out = pl.pallas_call(
        _stage_d_body,
        grid=(n // bn,),
        in_specs=in_specs,
        out_specs=pl.BlockSpec((bn, dpad), lambda i: (i, 0)),
        out_shape=jax.ShapeDtypeStruct((n, dpad), jnp.float32),
    )(*args)
    return out[:, :d6out]


def _sc_edge_body(acc_rows, nblk, epw,
                  h4n, ea4, sidx, didx, t16, s_out, n_out,
                  src_v, dst_v, r0, r1, r2, a0, a1, a2, m0, m1, m2,
                  tv_v, s_acc, n_acc,
                  g0, g1, g2, s0, s1, s2):
    rows = (r0, r1, r2)
    eav = (a0, a1, a2)
    mev = (m0, m1, m2)
    gsem = (g0, g1, g2)
    ssem = (s0, s1, s2)
    c = lax.axis_index("c")
    sid = lax.axis_index("s")
    stripe = acc_rows // NS
    zeros16 = jnp.zeros((16,), jnp.float32)

    pltpu.sync_copy(didx.at[sid], dst_v)
    pltpu.sync_copy(t16, tv_v)
    tv = tv_v[...]
    edge_base = sid * epw

    for p in range(2):
        q = c * 2 + p

        def zrow(r, carry):
            for k in range(HQ // 16):
                rows[0][r, pl.ds(k * 16, 16)] = zeros16
            return carry
        lax.fori_loop(0, B, zrow, 0, unroll=4)

        def zchunk(k, carry):
            base = sid * stripe + k * B
            pltpu.sync_copy(rows[0], s_acc.at[pl.ds(base, B)])
            pltpu.sync_copy(rows[0], n_acc.at[pl.ds(base, B)])
            return carry
        lax.fori_loop(0, stripe // B, zchunk, 0)

        pltpu.sync_copy(sidx.at[c, p, sid], src_v)
        plsc.subcore_barrier()

        def issue_in(j, b):
            pltpu.async_copy(h4n.at[src_v.at[j]], rows[b], gsem[b])
            pltpu.async_copy(
                ea4.at[q, pl.ds((edge_base + j * B) // 4, B // 4)],
                eav[b], gsem[b])

        def wait_in(b):
            for _ in range(2):
                pltpu.make_async_copy(ea4.at[q, pl.ds(0, B // 4)], eav[b],
                                      gsem[b]).wait()

        def compute(b):
            rb, ab, mb_ = rows[b], eav[b], mev[b]
            nch = HQ // 16

            def rowfn(g, carry):
                r0 = g * 4
                sls = [(r0 + u, pl.ds(k * 16, 16), pl.ds(u * HQ + k * 16, 16))
                       for u in range(4) for k in range(nch)]
                hv = [rb[r, sl] for r, sl, _ in sls]
                av = [ab[g, asl] for _, _, asl in sls]
                ms = [jnp.maximum(h + a, 0.0) + EPS
                      for h, a in zip(hv, av)]
                es = [jnp.exp(m * tv) for m in ms]
                mes = [m * e for m, e in zip(ms, es)]
                for (r, sl, _), e in zip(sls, es):
                    rb[r, sl] = e
                for (r, sl, _), me in zip(sls, mes):
                    mb_[r, sl] = me
                return carry
            lax.fori_loop(0, B // 4, rowfn, 0)

        def issue_scatter(j, b):
            pltpu.async_copy(rows[b], s_acc.at[dst_v.at[j]], ssem[b],
                             add=True)
            pltpu.async_copy(mev[b], n_acc.at[dst_v.at[j]], ssem[b],
                             add=True)

        def wait_scatter(b):
            for _ in range(2):
                pltpu.make_async_copy(ea4.at[q, pl.ds(0, B // 4)], eav[b],
                                      ssem[b]).wait()

        issue_in(0, 0)
        issue_in(1, 1)
        wait_in(0)
        compute(0)
        issue_scatter(0, 0)
        issue_in(2, 2)

        @pl.loop(1, nblk - 2, step=3)
        def _main(j0):
            for db in range(3):
                j = j0 + db
                b = (1 + db) % 3
                wait_in(b)
                compute(b)
                issue_scatter(j, b)
                bp = db % 3
                wait_scatter(bp)
                issue_in(j + 2, bp)

        wait_in(1)
        compute(1)
        issue_scatter(nblk - 2, 1)
        wait_in(2)
        compute(2)
        issue_scatter(nblk - 1, 2)
        for b in range(3):
            wait_scatter(b)
        plsc.subcore_barrier()

        pltpu.sync_copy(s_acc.at[pl.ds(sid * stripe, stripe)],
                        s_out.at[q, pl.ds(sid * stripe, stripe)])
        pltpu.sync_copy(n_acc.at[pl.ds(sid * stripe, stripe)],
                        n_out.at[q, pl.ds(sid * stripe, stripe)])


def _sc_edge(h4n, ea4, sidx, didx, t16, acc_rows, nblk, epw):
    mesh = plsc.VectorSubcoreMesh(core_axis_name="c", subcore_axis_name="s")
    body = functools.partial(_sc_edge_body, acc_rows, nblk, epw)
    f = pl.kernel(
        body,
        out_type=[
            jax.ShapeDtypeStruct((NQ, acc_rows, HQ), jnp.float32),
            jax.ShapeDtypeStruct((NQ, acc_rows, HQ), jnp.float32),
        ],
        mesh=mesh,
        compiler_params=pltpu.CompilerParams(use_tc_tiling_on_sc=False),
        scratch_types=[
            pltpu.VMEM((nblk, B), jnp.int32),
            pltpu.VMEM((nblk, B), jnp.int32),
            pltpu.VMEM((B, HQ), jnp.float32),
            pltpu.VMEM((B, HQ), jnp.float32),
            pltpu.VMEM((B, HQ), jnp.float32),
            pltpu.VMEM((B // 4, 4 * HQ), jnp.float32),
            pltpu.VMEM((B // 4, 4 * HQ), jnp.float32),
            pltpu.VMEM((B // 4, 4 * HQ), jnp.float32),
            pltpu.VMEM((B, HQ), jnp.float32),
            pltpu.VMEM((B, HQ), jnp.float32),
            pltpu.VMEM((B, HQ), jnp.float32),
            pltpu.VMEM((16,), jnp.float32),
            pltpu.VMEM_SHARED((acc_rows, HQ), jnp.float32),
            pltpu.VMEM_SHARED((acc_rows, HQ), jnp.float32),
            pltpu.SemaphoreType.DMA,
            pltpu.SemaphoreType.DMA,
            pltpu.SemaphoreType.DMA,
            pltpu.SemaphoreType.DMA,
            pltpu.SemaphoreType.DMA,
            pltpu.SemaphoreType.DMA,
        ],
    )
    return f(h4n, ea4, sidx, didx, t16)


def kernel(node_features, edge_index, edge_features, params):
    x = node_features
    n, d = x.shape
    e, ed = edge_features.shape

    nblk = (e + NS * B - 1) // (NS * B)
    nblk = ((nblk + 2) // 3) * 3
    epw = nblk * B
    e_pad = NS * epw
    acc_rows = ((n + NS * B - 1) // (NS * B)) * (NS * B)
    trash = acc_rows - 1

    src = edge_index[0]
    dst = edge_index[1]
    pad = e_pad - e
    src_pad = jnp.concatenate([src, jnp.zeros((pad,), jnp.int32)])
    dst_pad = jnp.concatenate([dst, jnp.full((pad,), trash, jnp.int32)])
    sidx = jnp.stack([jnp.stack([src_pad + (2 * c + p) * n for p in range(2)])
                      for c in range(NC)]).reshape(NC, 2, NS, nblk, B)
    didx = dst_pad.reshape(NS, nblk, B)

    stacked = jax.tree.map(lambda *xs: jnp.stack(xs), *params['gcn'])

    eattr_r = edge_features.reshape(e // 4, 4 * ed)

    def layer_step(xc, p):
        h4 = _stage_a(xc, p['lg'], p['lb'])
        we = p['We']
        w_pack = jnp.zeros((4 * ed, NQ * 4 * HQ), jnp.float32)
        b_pack = jnp.zeros((NQ * 4 * HQ,), jnp.float32)
        for u in range(4):
            for qq in range(NQ):
                w_pack = w_pack.at[
                    u * ed:(u + 1) * ed,
                    qq * 4 * HQ + u * HQ:qq * 4 * HQ + (u + 1) * HQ].set(
                        we[:, qq * HQ:(qq + 1) * HQ])
                b_pack = b_pack.at[
                    qq * 4 * HQ + u * HQ:qq * 4 * HQ + (u + 1) * HQ].set(
                        p['be'][qq * HQ:(qq + 1) * HQ])
        ea4 = _stage_b(eattr_r, w_pack, b_pack, e_pad)
        t16 = jnp.broadcast_to(p['t'], (16,)).astype(jnp.float32)
        s_acc, n_acc = _sc_edge(h4.reshape(NQ * n, HQ), ea4, sidx, didx,
                                t16, acc_rows, nblk, epw)
        xc = _stage_c(s_acc, n_acc, h4, xc, p['W1'], p['b1'], p['mg'],
                      p['mb'], p['W2'], p['b2'])
        return xc, None

    x, _ = lax.scan(layer_step, x, stacked)

    return _stage_d(x, params['mlp'])
